# baseline (speedup 1.0000x reference)
import jax
import jax.numpy as jnp
import numpy as np
from jax.experimental import pallas as pl
from jax.experimental.pallas import tpu as pltpu

L0, L1, L2 = 64, 32, 16
G, NPG = 100, 100
N = G * NPG
D = L0 + 3 * L1 + 5 * L2
B = 10
PAD = 128

_PREC = jax.lax.Precision.DEFAULT


def _dot(a, b):
    return jnp.dot(a, b, precision=_PREC, preferred_element_type=jnp.float32)


def _block_kernel(x_ref, pos_ref, posT_ref,
                  A_ref, W0_ref, P1_ref, P2_ref,
                  T1_ref, T2_ref, Wg1_ref, Wg2_ref,
                  Wr_ref, br_ref, o_ref):
    x = x_ref[:]

    Y = _dot(x, A_ref[:])
    p1f = _dot(Y[:, :L0], P1_ref[:])
    p2f = _dot(Y[:, :L0], P2_ref[:])

    WrA = Wr_ref[:]
    brA = br_ref[:]
    ii = jax.lax.broadcasted_iota(jnp.int32, (NPG, NPG), 0)
    jj = jax.lax.broadcasted_iota(jnp.int32, (NPG, NPG), 1)
    mask = jnp.where(ii != jj, 1.0 / np.sqrt(NPG), 0.0).astype(jnp.float32)
    s3 = float(np.sqrt(3.0))

    c0l, c1al, c2al = [], [], []
    c1bl = [[] for _ in range(3)]
    c2bl = [[] for _ in range(5)]
    for q in range(B):
        sl = slice(q * NPG, (q + 1) * NPG)
        rel = [posT_ref[k:k + 1, q * PAD:q * PAD + NPG] - pos_ref[sl, k:k + 1]
               for k in range(3)]
        d2 = rel[0] * rel[0] + rel[1] * rel[1] + rel[2] * rel[2] + 1e-9
        dist = jnp.sqrt(d2)
        inv = 1.0 / dist
        rx, ry, rz = rel[0] * inv, rel[1] * inv, rel[2] * inv

        wch = [None] * 5
        for b in range(8):
            c = 4.0 * b / 7.0
            g = dist - c
            bas = jnp.exp(g * g * -2.0)
            for m in range(5):
                t = bas * WrA[b:b + 1, m:m + 1]
                wch[m] = t if wch[m] is None else wch[m] + t
        for m in range(5):
            wch[m] = (wch[m] + brA[0:1, m:m + 1]) * mask

        Y1 = [rx, ry, rz]
        Y2 = [s3 * rx * ry, s3 * ry * rz, 0.5 * (3.0 * rz * rz - 1.0),
              s3 * rx * rz, 0.5 * s3 * (rx * rx - ry * ry)]

        c0l.append(_dot(wch[0], Y[sl, :L0]))
        c1al.append(_dot(wch[1], Y[sl, L0:L0 + 3 * L1]))
        c2al.append(_dot(wch[2], Y[sl, L0 + 3 * L1:]))
        for k in range(3):
            c1bl[k].append(_dot(wch[3] * Y1[k], p1f[sl]))
        for m in range(5):
            c2bl[m].append(_dot(wch[4] * Y2[m], p2f[sl]))

    c0 = jnp.concatenate(c0l, axis=0)
    c1a = jnp.concatenate(c1al, axis=0)
    c2a = jnp.concatenate(c2al, axis=0)

    T1 = T1_ref[:]
    T2 = T2_ref[:]
    o1 = _dot(c1a, T1)
    for k in range(3):
        o1 = o1 + _dot(jnp.concatenate(c1bl[k], axis=0),
                       T1[k * L1:(k + 1) * L1])
    o2 = _dot(c2a, T2)
    for m in range(5):
        o2 = o2 + _dot(jnp.concatenate(c2bl[m], axis=0),
                       T2[m * L2:(m + 1) * L2])
    m0 = x[:, :L0] + _dot(c0, W0_ref[:])
    g1 = jax.nn.sigmoid(_dot(m0, Wg1_ref[:]))
    g2 = jax.nn.sigmoid(_dot(m0, Wg2_ref[:]))
    o_ref[:] = jnp.concatenate(
        [m0 * jax.nn.sigmoid(m0),
         (x[:, L0:L0 + 3 * L1] + o1) * g1,
         (x[:, L0 + 3 * L1:] + o2) * g2], axis=1)


def kernel(nodes, pos, batch, W0, W1, W2, P1, P2, Wr, br, Wg1, Wg2):
    del batch
    posTg = pos.reshape(G, NPG, 3).transpose(0, 2, 1)
    posT = jnp.pad(posTg, ((0, 0), (0, 0), (0, PAD - NPG)))
    posT = posT.transpose(1, 0, 2).reshape(3, G * PAD)
    br2 = br.reshape(1, 5)

    I3 = jnp.eye(3, dtype=jnp.float32)
    I5 = jnp.eye(5, dtype=jnp.float32)
    A = jnp.zeros((D, D), dtype=jnp.float32)
    A = A.at[:L0, :L0].set(W0)
    A = A.at[L0:L0 + 3 * L1, L0:L0 + 3 * L1].set(
        jnp.einsum('cd,ik->cikd', W1, I3).reshape(3 * L1, 3 * L1))
    A = A.at[L0 + 3 * L1:, L0 + 3 * L1:].set(
        jnp.einsum('cd,ik->cikd', W2, I5).reshape(5 * L2, 5 * L2))
    T1cat = jnp.einsum('cd,kq->kcdq', W1, I3).reshape(3 * L1, 3 * L1)
    T2cat = jnp.einsum('cd,kq->kcdq', W2, I5).reshape(5 * L2, 5 * L2)
    Wg1rep = jnp.einsum('ac,k->ack', Wg1, jnp.ones(3)).reshape(L0, 3 * L1)
    Wg2rep = jnp.einsum('ac,k->ack', Wg2, jnp.ones(5)).reshape(L0, 5 * L2)

    full = lambda shape: pl.BlockSpec(shape, lambda g: (0,) * len(shape))
    out = pl.pallas_call(
        _block_kernel,
        grid=(G // B,),
        in_specs=[
            pl.BlockSpec((B * NPG, D), lambda g: (g, 0)),
            pl.BlockSpec((B * NPG, 3), lambda g: (g, 0)),
            pl.BlockSpec((3, B * PAD), lambda g: (0, g)),
            full((D, D)), full((L0, L0)), full((L0, L1)), full((L0, L2)),
            full((3 * L1, 3 * L1)), full((5 * L2, 5 * L2)),
            full((L0, 3 * L1)), full((L0, 5 * L2)),
            full((8, 5)), full((1, 5)),
        ],
        out_specs=pl.BlockSpec((B * NPG, D), lambda g: (g, 0)),
        out_shape=jax.ShapeDtypeStruct((N, D), jnp.float32),
        compiler_params=pltpu.CompilerParams(
            dimension_semantics=("arbitrary",)),
    )(nodes, pos, posT, A, W0, P1, P2, T1cat, T2cat, Wg1rep, Wg2rep, Wr, br2)

    return out


# speedup vs baseline: 1.0775x; 1.0775x over previous
import jax
import jax.numpy as jnp
import numpy as np
from jax.experimental import pallas as pl
from jax.experimental.pallas import tpu as pltpu

L0, L1, L2 = 64, 32, 16
G, NPG = 100, 100
N = G * NPG
D = L0 + 3 * L1 + 5 * L2
B = 20
PAD = 128

_PREC = jax.lax.Precision.DEFAULT


def _dot(a, b):
    return jnp.dot(a, b, precision=_PREC, preferred_element_type=jnp.float32)


def _block_kernel(x_ref, pos_ref, posT_ref,
                  A_ref, W0_ref, P1_ref, P2_ref,
                  T1_ref, T2_ref, Wg1_ref, Wg2_ref,
                  Wr_ref, br_ref, o_ref):
    x = x_ref[:]

    Y = _dot(x, A_ref[:])
    p1f = _dot(Y[:, :L0], P1_ref[:])
    p2f = _dot(Y[:, :L0], P2_ref[:])

    WrA = Wr_ref[:]
    brA = br_ref[:]
    ii = jax.lax.broadcasted_iota(jnp.int32, (NPG, NPG), 0)
    jj = jax.lax.broadcasted_iota(jnp.int32, (NPG, NPG), 1)
    mask = jnp.where(ii != jj, 1.0 / np.sqrt(NPG), 0.0).astype(jnp.float32)
    s3 = float(np.sqrt(3.0))

    c0l, c1al, c2al = [], [], []
    c1bl = [[] for _ in range(3)]
    c2bl = [[] for _ in range(5)]
    for q in range(B):
        sl = slice(q * NPG, (q + 1) * NPG)
        rel = [posT_ref[k:k + 1, q * PAD:q * PAD + NPG] - pos_ref[sl, k:k + 1]
               for k in range(3)]
        d2 = rel[0] * rel[0] + rel[1] * rel[1] + rel[2] * rel[2] + 1e-9
        dist = jnp.sqrt(d2)
        inv = 1.0 / dist
        rx, ry, rz = rel[0] * inv, rel[1] * inv, rel[2] * inv

        wch = [None] * 5
        for b in range(8):
            c = 4.0 * b / 7.0
            g = dist - c
            bas = jnp.exp(g * g * -2.0)
            for m in range(5):
                t = bas * WrA[b:b + 1, m:m + 1]
                wch[m] = t if wch[m] is None else wch[m] + t
        for m in range(5):
            wch[m] = (wch[m] + brA[0:1, m:m + 1]) * mask

        Y1 = [rx, ry, rz]
        Y2 = [s3 * rx * ry, s3 * ry * rz, 0.5 * (3.0 * rz * rz - 1.0),
              s3 * rx * rz, 0.5 * s3 * (rx * rx - ry * ry)]

        c0l.append(_dot(wch[0], Y[sl, :L0]))
        c1al.append(_dot(wch[1], Y[sl, L0:L0 + 3 * L1]))
        c2al.append(_dot(wch[2], Y[sl, L0 + 3 * L1:]))
        for k in range(3):
            c1bl[k].append(_dot(wch[3] * Y1[k], p1f[sl]))
        for m in range(5):
            c2bl[m].append(_dot(wch[4] * Y2[m], p2f[sl]))

    c0 = jnp.concatenate(c0l, axis=0)
    c1a = jnp.concatenate(c1al, axis=0)
    c2a = jnp.concatenate(c2al, axis=0)

    T1 = T1_ref[:]
    T2 = T2_ref[:]
    o1 = _dot(c1a, T1)
    for k in range(3):
        o1 = o1 + _dot(jnp.concatenate(c1bl[k], axis=0),
                       T1[k * L1:(k + 1) * L1])
    o2 = _dot(c2a, T2)
    for m in range(5):
        o2 = o2 + _dot(jnp.concatenate(c2bl[m], axis=0),
                       T2[m * L2:(m + 1) * L2])
    m0 = x[:, :L0] + _dot(c0, W0_ref[:])
    g1 = jax.nn.sigmoid(_dot(m0, Wg1_ref[:]))
    g2 = jax.nn.sigmoid(_dot(m0, Wg2_ref[:]))
    o_ref[:] = jnp.concatenate(
        [m0 * jax.nn.sigmoid(m0),
         (x[:, L0:L0 + 3 * L1] + o1) * g1,
         (x[:, L0 + 3 * L1:] + o2) * g2], axis=1)


def kernel(nodes, pos, batch, W0, W1, W2, P1, P2, Wr, br, Wg1, Wg2):
    del batch
    posTg = pos.reshape(G, NPG, 3).transpose(0, 2, 1)
    posT = jnp.pad(posTg, ((0, 0), (0, 0), (0, PAD - NPG)))
    posT = posT.transpose(1, 0, 2).reshape(3, G * PAD)
    br2 = br.reshape(1, 5)

    I3 = jnp.eye(3, dtype=jnp.float32)
    I5 = jnp.eye(5, dtype=jnp.float32)
    A = jnp.zeros((D, D), dtype=jnp.float32)
    A = A.at[:L0, :L0].set(W0)
    A = A.at[L0:L0 + 3 * L1, L0:L0 + 3 * L1].set(
        jnp.einsum('cd,ik->cikd', W1, I3).reshape(3 * L1, 3 * L1))
    A = A.at[L0 + 3 * L1:, L0 + 3 * L1:].set(
        jnp.einsum('cd,ik->cikd', W2, I5).reshape(5 * L2, 5 * L2))
    T1cat = jnp.einsum('cd,kq->kcdq', W1, I3).reshape(3 * L1, 3 * L1)
    T2cat = jnp.einsum('cd,kq->kcdq', W2, I5).reshape(5 * L2, 5 * L2)
    Wg1rep = jnp.einsum('ac,k->ack', Wg1, jnp.ones(3)).reshape(L0, 3 * L1)
    Wg2rep = jnp.einsum('ac,k->ack', Wg2, jnp.ones(5)).reshape(L0, 5 * L2)

    full = lambda shape: pl.BlockSpec(shape, lambda g: (0,) * len(shape))
    out = pl.pallas_call(
        _block_kernel,
        grid=(G // B,),
        in_specs=[
            pl.BlockSpec((B * NPG, D), lambda g: (g, 0)),
            pl.BlockSpec((B * NPG, 3), lambda g: (g, 0)),
            pl.BlockSpec((3, B * PAD), lambda g: (0, g)),
            full((D, D)), full((L0, L0)), full((L0, L1)), full((L0, L2)),
            full((3 * L1, 3 * L1)), full((5 * L2, 5 * L2)),
            full((L0, 3 * L1)), full((L0, 5 * L2)),
            full((8, 5)), full((1, 5)),
        ],
        out_specs=pl.BlockSpec((B * NPG, D), lambda g: (g, 0)),
        out_shape=jax.ShapeDtypeStruct((N, D), jnp.float32),
        compiler_params=pltpu.CompilerParams(
            dimension_semantics=("arbitrary",)),
    )(nodes, pos, posT, A, W0, P1, P2, T1cat, T2cat, Wg1rep, Wg2rep, Wr, br2)

    return out
